# Initial kernel scaffold; baseline (speedup 1.0000x reference)
import jax
import jax.numpy as jnp
from jax.experimental import pallas as pl


def kernel(x, params, edge_index, batch):
    raise NotImplementedError("write your pallas kernel here")


# speedup vs baseline: 4.2989x; 4.2989x over previous
import functools

import jax
import jax.numpy as jnp
from jax import lax
from jax.experimental import pallas as pl
from jax.experimental.pallas import tpu as pltpu
from jax.experimental.pallas import tpu_sc as plsc

_N = 10000
_E = 320000
_D = 128
_H = 128
_G = 128
_NL = 3

_NSC = 2
_NTILES = 32
_EPT = _E // _NTILES
_C = 80
_NCHUNK = _EPT // _C
_RPT = 624
_TAIL = _N - 16 * _RPT


def _sc_scatter_body(src_hbm, dst_hbm, h_hbm, zero_hbm, out_hbm,
                     sidx, didx, rows, agg_sh, sem):
    c = lax.axis_index("c")
    s = lax.axis_index("s")
    pltpu.sync_copy(zero_hbm.at[pl.ds(s * _RPT, _RPT)],
                    agg_sh.at[pl.ds(s * _RPT, _RPT)])

    @pl.when(s == 15)
    def _():
        pltpu.sync_copy(zero_hbm.at[pl.ds(16 * _RPT, _TAIL)],
                        agg_sh.at[pl.ds(16 * _RPT, _TAIL)])

    plsc.subcore_barrier()

    base = (c * 16 + s) * _EPT

    def body(i, carry):
        off = base + i * _C
        pltpu.sync_copy(src_hbm.at[pl.ds(off, _C)], sidx)
        pltpu.sync_copy(dst_hbm.at[pl.ds(off, _C)], didx)
        pltpu.async_copy(h_hbm.at[sidx], rows, sem).wait()
        pltpu.sync_copy(rows, agg_sh.at[didx], add=True)
        return carry

    lax.fori_loop(0, _NCHUNK, body, 0)
    plsc.subcore_barrier()
    pltpu.sync_copy(agg_sh.at[pl.ds(s * _RPT, _RPT)],
                    out_hbm.at[c, pl.ds(s * _RPT, _RPT)])

    @pl.when(s == 15)
    def _():
        pltpu.sync_copy(agg_sh.at[pl.ds(16 * _RPT, _TAIL)],
                        out_hbm.at[c, pl.ds(16 * _RPT, _TAIL)])


_sc_scatter = pl.kernel(
    _sc_scatter_body,
    out_type=jax.ShapeDtypeStruct((_NSC, _N, _D), jnp.float32),
    mesh=plsc.VectorSubcoreMesh(core_axis_name="c", subcore_axis_name="s"),
    scratch_types=[
        pltpu.VMEM((_C,), jnp.int32),
        pltpu.VMEM((_C,), jnp.int32),
        pltpu.VMEM((_C, _D), jnp.float32),
        pltpu.VMEM_SHARED((_N, _D), jnp.float32),
        pltpu.SemaphoreType.DMA,
    ],
)


def _matT(a, w):
    return lax.dot_general(a, w, (((1,), (1,)), ((), ())),
                           preferred_element_type=jnp.float32)


def _bn_cols(t, g, b):
    m = jnp.mean(t, axis=0, keepdims=True)
    v = jnp.mean(t * t, axis=0, keepdims=True) - m * m
    return g * (t - m) * lax.rsqrt(v + 1e-5) + b


def _lin_body(x_ref, w_ref, b_ref, o_ref):
    o_ref[...] = jnp.maximum(_matT(x_ref[...], w_ref[...]) + b_ref[...], 0.0)


_lin = pl.pallas_call(
    _lin_body, out_shape=jax.ShapeDtypeStruct((_N, _H), jnp.float32))


def _gin_body(h_ref, p0_ref, p1_ref, w1_ref, b1_ref, g1_ref, be1_ref,
              w2_ref, b2_ref, g2_ref, be2_ref, lng_ref, lnb_ref, o_ref):
    z = h_ref[...] + p0_ref[...] + p1_ref[...]
    t = _matT(z, w1_ref[...]) + b1_ref[...]
    t = jnp.maximum(_bn_cols(t, g1_ref[...], be1_ref[...]), 0.0)
    t = _matT(t, w2_ref[...]) + b2_ref[...]
    t = jnp.maximum(_bn_cols(t, g2_ref[...], be2_ref[...]), 0.0)
    m = jnp.mean(t, axis=1, keepdims=True)
    v = jnp.mean(t * t, axis=1, keepdims=True) - m * m
    o_ref[...] = lng_ref[...] * (t - m) * lax.rsqrt(v + 1e-5) + lnb_ref[...]


_gin = pl.pallas_call(
    _gin_body, out_shape=jax.ShapeDtypeStruct((_N, _H), jnp.float32))


def _segmax_body(b_ref, h_ref, o_ref):
    g = pl.program_id(0)
    bm = b_ref[...]
    start = jnp.sum(jnp.where(bm < g, 1, 0))
    end = start + jnp.sum(jnp.where(bm == g, 1, 0))
    k0 = start // 8
    k1 = (end + 7) // 8

    def body(k, acc):
        rows = h_ref[pl.ds(k * 8, 8), :]
        rid = k * 8 + lax.broadcasted_iota(jnp.int32, (8, _D), 0)
        valid = (rid >= start) & (rid < end)
        return jnp.maximum(acc, jnp.where(valid, rows, -jnp.inf))

    acc = lax.fori_loop(k0, k1, body,
                        jnp.full((8, _D), -jnp.inf, jnp.float32))
    o_ref[...] = jnp.max(acc, axis=0, keepdims=True)[None]


_segmax = pl.pallas_call(
    _segmax_body,
    grid=(_G,),
    in_specs=[pl.BlockSpec((80, 128), lambda g: (0, 0)),
              pl.BlockSpec((_N, _D), lambda g: (0, 0))],
    out_specs=pl.BlockSpec((1, 1, _D), lambda g: (g, 0, 0)),
    out_shape=jax.ShapeDtypeStruct((_G, 1, _D), jnp.float32),
)


def _head_body(p_ref, w1_ref, b1_ref, g1_ref, be1_ref,
               w2_ref, b2_ref, g2_ref, be2_ref, w3_ref, b3_ref, o_ref):
    t = jnp.maximum(_matT(p_ref[...], w1_ref[...]) + b1_ref[...], 0.0)
    t = _bn_cols(t, g1_ref[...], be1_ref[...])
    t = jnp.maximum(_matT(t, w2_ref[...]) + b2_ref[...], 0.0)
    t = _bn_cols(t, g2_ref[...], be2_ref[...])
    t = jnp.sum(t * w3_ref[...], axis=1, keepdims=True) + b3_ref[0, 0]
    o_ref[...] = jax.nn.sigmoid(t)


_head = pl.pallas_call(
    _head_body, out_shape=jax.ShapeDtypeStruct((_G, 1), jnp.float32))


def kernel(x, params, edge_index, batch):
    ei = edge_index.astype(jnp.int32)
    src = ei[0]
    dst = ei[1]
    bpad = jnp.concatenate(
        [batch.astype(jnp.int32),
         jnp.full((80 * 128 - _N,), _G, jnp.int32)]).reshape(80, 128)
    zero = jnp.zeros((_N, _D), jnp.float32)

    def row(v):
        return v.reshape(1, -1)

    h = _lin(x, params['lin_W'], row(params['lin_b']))
    for i in range(_NL):
        parts = _sc_scatter(src, dst, h, zero)
        h = _gin(h, parts[0], parts[1],
                 params[f'l{i}_W1'], row(params[f'l{i}_b1']),
                 row(params[f'l{i}_g1']), row(params[f'l{i}_be1']),
                 params[f'l{i}_W2'], row(params[f'l{i}_b2']),
                 row(params[f'l{i}_g2']), row(params[f'l{i}_be2']),
                 row(params[f'l{i}_lng']), row(params[f'l{i}_lnb']))
    p = _segmax(bpad, h).reshape(_G, _D)
    y = _head(p, params['fc1_W'], row(params['fc1_b']),
              row(params['bn1_g']), row(params['bn1_b']),
              params['fc2_W'], row(params['fc2_b']),
              row(params['bn2_g']), row(params['bn2_b']),
              params['fc3_W'], row(params['fc3_b']))
    return y
